# baseline (speedup 1.0000x reference)
import functools

import jax
import jax.numpy as jnp
from jax import lax
from jax.experimental import pallas as pl
from jax.experimental.pallas import tpu as pltpu
from jax.experimental.pallas import tpu_sc as plsc

NC = 2
NS = 16
NW = NC * NS
B = 128
GRP = 4

f32 = jnp.float32


def _mesh():
    return plsc.VectorSubcoreMesh(
        core_axis_name="c", subcore_axis_name="s", num_cores=NC, num_subcores=NS
    )


def _deg_call(dst_r, zeros, ones, n_pad, ch):
    rps = n_pad // NS

    @functools.partial(
        pl.kernel,
        out_type=jax.ShapeDtypeStruct((NC, n_pad, 16), f32),
        mesh=_mesh(),
        scratch_types=[
            pltpu.VMEM((ch, B), jnp.int32),
            pltpu.VMEM((B, 16), f32),
            pltpu.VMEM_SHARED((n_pad, 16), f32),
            pltpu.SemaphoreType.DMA,
        ],
        compiler_params=pltpu.CompilerParams(use_tc_tiling_on_sc=False),
    )
    def k(dst_hbm, zeros_hbm, ones_hbm, out_hbm, dst_v, ones_v, acc_sh, dsem):
        c = lax.axis_index("c")
        s = lax.axis_index("s")
        w = c * NS + s
        pltpu.sync_copy(zeros_hbm.at[pl.ds(s * rps, rps)],
                        acc_sh.at[pl.ds(s * rps, rps)])
        pltpu.sync_copy(ones_hbm, ones_v)
        pltpu.sync_copy(dst_hbm.at[w], dst_v)
        plsc.subcore_barrier()

        def body(j, carry):
            pltpu.async_copy(ones_v, acc_sh.at[dst_v.at[j]], dsem, add=True)
            return carry

        lax.fori_loop(0, ch, body, 0)

        def drain(j, carry):
            pltpu.make_async_copy(ones_v, acc_sh.at[dst_v.at[0]], dsem).wait()
            return carry

        lax.fori_loop(0, ch, drain, 0)
        plsc.subcore_barrier()
        pltpu.sync_copy(acc_sh.at[pl.ds(s * rps, rps)],
                        out_hbm.at[c, pl.ds(s * rps, rps)])

    return k(dst_r, zeros, ones)


def _agg_call(src_r, dst_r, table, n_pad, ch, w_feat):
    rps = n_pad // NS
    slots = 2 * GRP
    ngrp = ch // GRP

    @functools.partial(
        pl.kernel,
        out_type=jax.ShapeDtypeStruct((NC, n_pad, w_feat), f32),
        mesh=_mesh(),
        scratch_types=[
            pltpu.VMEM((ch, B), jnp.int32),
            pltpu.VMEM((ch, B), jnp.int32),
            pltpu.VMEM((slots, B, w_feat), f32),
            pltpu.VMEM_SHARED((n_pad, w_feat), f32),
            pltpu.SemaphoreType.DMA((slots,)),
            pltpu.SemaphoreType.DMA((slots,)),
        ],
        compiler_params=pltpu.CompilerParams(use_tc_tiling_on_sc=False),
    )
    def k(src_hbm, dst_hbm, table_hbm, out_hbm, src_v, dst_v, rows_v, acc_sh,
          gsem, ssem):
        c = lax.axis_index("c")
        s = lax.axis_index("s")
        w = c * NS + s
        pltpu.sync_copy(src_hbm.at[w], src_v)
        pltpu.sync_copy(dst_hbm.at[w], dst_v)
        pltpu.sync_copy(table_hbm.at[pl.ds(s * rps, rps)],
                        acc_sh.at[pl.ds(s * rps, rps)])
        plsc.subcore_barrier()

        def gissue(j, slot):
            pltpu.async_copy(table_hbm.at[src_v.at[j]], rows_v.at[slot],
                             gsem.at[slot])

        def gwait(slot):
            pltpu.make_async_copy(table_hbm.at[src_v.at[0]], rows_v.at[slot],
                                  gsem.at[slot]).wait()

        def sissue(j, slot):
            pltpu.async_copy(rows_v.at[slot], acc_sh.at[dst_v.at[j]],
                             ssem.at[slot], add=True)

        def swait(slot):
            pltpu.make_async_copy(rows_v.at[slot], acc_sh.at[dst_v.at[0]],
                                  ssem.at[slot]).wait()

        for b in range(GRP):
            gissue(b, b)

        def body(g, carry):
            h = g % 2
            base = h * GRP
            ob = (1 - h) * GRP
            for b in range(GRP):
                gwait(base + b)
            for b in range(GRP):
                sissue(g * GRP + b, base + b)

            @pl.when(g + 1 < ngrp)
            def _():
                for b in range(GRP):
                    @pl.when(g >= 1)
                    def _():
                        swait(ob + b)
                    gissue((g + 1) * GRP + b, ob + b)
            return carry

        lax.fori_loop(0, ngrp, body, 0)
        for b in range(slots):
            swait(b)
        plsc.subcore_barrier()
        pltpu.sync_copy(acc_sh.at[pl.ds(s * rps, rps)],
                        out_hbm.at[c, pl.ds(s * rps, rps)])

    return k(src_r, dst_r, table)


def _tc_b_call(x_pad, w1, b1, degs, n_pad, hid):
    def body(x_ref, w_ref, b_ref, deg_ref, t1_ref, norm_ref):
        deg = deg_ref[0] + deg_ref[1]
        norm = lax.rsqrt(deg[:, 0:1] + 1.0)
        h1 = jnp.dot(x_ref[...], w_ref[...],
                     preferred_element_type=f32) + b_ref[...]
        t1_ref[...] = h1 * norm
        norm_ref[...] = jnp.broadcast_to(norm, (n_pad, 16))

    return pl.pallas_call(
        body,
        out_shape=(
            jax.ShapeDtypeStruct((n_pad, hid), f32),
            jax.ShapeDtypeStruct((n_pad, 16), f32),
        ),
    )(x_pad, w1, b1, degs)


def _tc_c_call(acc1, t1, norm16, w2, b2, n_pad, f_out):
    def body(acc_ref, t1_ref, norm_ref, w_ref, b_ref, t2_ref):
        norm = norm_ref[:, 0:1]
        agg1 = norm * (acc_ref[0] + acc_ref[1] - t1_ref[...])
        g = jnp.where(agg1 > 0.0,
                      agg1, jnp.exp(jnp.minimum(agg1, 0.0)) - 1.0)
        h2 = jnp.dot(g, w_ref[...], preferred_element_type=f32) + b_ref[...]
        t2_ref[...] = h2 * norm

    return pl.pallas_call(
        body,
        out_shape=jax.ShapeDtypeStruct((n_pad, f_out), f32),
    )(acc1, t1, norm16, w2, b2)


def _tc_d_call(acc2, t2, norm16, n, f_out):
    def body(acc_ref, t2_ref, norm_ref, out_ref):
        norm = norm_ref[:, 0:1]
        agg2 = norm * (acc_ref[0] + acc_ref[1] - t2_ref[...])
        a = agg2[:n]
        m = jnp.max(a, axis=1, keepdims=True)
        lse = jnp.log(jnp.sum(jnp.exp(a - m), axis=1, keepdims=True))
        out_ref[...] = a - m - lse

    return pl.pallas_call(
        body,
        out_shape=jax.ShapeDtypeStruct((n, f_out), f32),
    )(acc2, t2, norm16)


def kernel(x, edge_index, W1, b1, W2, b2):
    n, f_in = x.shape
    hid = W1.shape[1]
    f_out = W2.shape[1]
    e = edge_index.shape[1]

    align = NS * 8
    n_pad = ((n + 1 + align - 1) // align) * align
    ch = -(-e // (NW * B))
    ch = max(2 * GRP, ((ch + GRP - 1) // GRP) * GRP)
    e_pad = NW * ch * B

    pad = jnp.full((e_pad - e,), n, dtype=jnp.int32)
    src_r = jnp.concatenate([edge_index[0], pad]).reshape(NW, ch, B)
    dst_r = jnp.concatenate([edge_index[1], pad]).reshape(NW, ch, B)

    x_pad = jnp.zeros((n_pad, f_in), f32).at[:n].set(x)
    zeros = jnp.zeros((n_pad, 16), f32)
    ones = jnp.ones((B, 16), f32)

    degs = _deg_call(dst_r, zeros, ones, n_pad, ch)
    t1, norm16 = _tc_b_call(x_pad, W1, b1.reshape(1, hid), degs, n_pad, hid)
    acc1 = _agg_call(src_r, dst_r, t1, n_pad, ch, hid)
    t2 = _tc_c_call(acc1, t1, norm16, W2, b2.reshape(1, f_out), n_pad, f_out)
    acc2 = _agg_call(src_r, dst_r, t2, n_pad, ch, f_out)
    return _tc_d_call(acc2, t2, norm16, n, f_out)


# speedup vs baseline: 1.9353x; 1.9353x over previous
import functools

import jax
import jax.numpy as jnp
from jax import lax
from jax.experimental import pallas as pl
from jax.experimental.pallas import tpu as pltpu
from jax.experimental.pallas import tpu_sc as plsc

NC = 2
NS = 16
NW = NC * NS
B = 128
GRP = 4

f32 = jnp.float32


def _mesh():
    return plsc.VectorSubcoreMesh(
        core_axis_name="c", subcore_axis_name="s", num_cores=NC, num_subcores=NS
    )


def _deg_call(dst_r, zeros, ones, n_pad, ch):
    rps = n_pad // NS

    @functools.partial(
        pl.kernel,
        out_type=jax.ShapeDtypeStruct((NC, n_pad, 16), f32),
        mesh=_mesh(),
        scratch_types=[
            pltpu.VMEM((ch, B), jnp.int32),
            pltpu.VMEM((B, 16), f32),
            pltpu.VMEM_SHARED((n_pad, 16), f32),
            pltpu.SemaphoreType.DMA,
        ],
        compiler_params=pltpu.CompilerParams(use_tc_tiling_on_sc=False),
    )
    def k(dst_hbm, zeros_hbm, ones_hbm, out_hbm, dst_v, ones_v, acc_sh, dsem):
        c = lax.axis_index("c")
        s = lax.axis_index("s")
        w = c * NS + s
        pltpu.sync_copy(zeros_hbm.at[pl.ds(s * rps, rps)],
                        acc_sh.at[pl.ds(s * rps, rps)])
        pltpu.sync_copy(ones_hbm, ones_v)
        pltpu.sync_copy(dst_hbm.at[w], dst_v)
        plsc.subcore_barrier()

        def body(j, carry):
            pltpu.async_copy(ones_v, acc_sh.at[dst_v.at[j]], dsem, add=True)
            return carry

        lax.fori_loop(0, ch, body, 0)

        def drain(j, carry):
            pltpu.make_async_copy(ones_v, acc_sh.at[dst_v.at[0]], dsem).wait()
            return carry

        lax.fori_loop(0, ch, drain, 0)
        plsc.subcore_barrier()
        pltpu.sync_copy(acc_sh.at[pl.ds(s * rps, rps)],
                        out_hbm.at[c, pl.ds(s * rps, rps)])

    return k(dst_r, zeros, ones)


def _agg_call(src_r, dst_r, table, n_pad, ch, w_feat):
    rps = n_pad // NS
    slots = 2 * GRP
    ngrp = ch // GRP

    @functools.partial(
        pl.kernel,
        out_type=jax.ShapeDtypeStruct((NC, n_pad, w_feat), f32),
        mesh=_mesh(),
        scratch_types=[
            pltpu.VMEM((ch, B), jnp.int32),
            pltpu.VMEM((ch, B), jnp.int32),
            pltpu.VMEM((slots, B, w_feat), f32),
            pltpu.VMEM_SHARED((n_pad, w_feat), f32),
            pltpu.VMEM_SHARED((n_pad, w_feat), f32),
            pltpu.SemaphoreType.DMA((slots,)),
            pltpu.SemaphoreType.DMA((slots,)),
        ],
        compiler_params=pltpu.CompilerParams(use_tc_tiling_on_sc=False),
    )
    def k(src_hbm, dst_hbm, table_hbm, out_hbm, src_v, dst_v, rows_v, acc_sh,
          tab_sh, gsem, ssem):
        c = lax.axis_index("c")
        s = lax.axis_index("s")
        w = c * NS + s
        pltpu.sync_copy(src_hbm.at[w], src_v)
        pltpu.sync_copy(dst_hbm.at[w], dst_v)
        pltpu.sync_copy(table_hbm.at[pl.ds(s * rps, rps)],
                        tab_sh.at[pl.ds(s * rps, rps)])
        pltpu.sync_copy(table_hbm.at[pl.ds(s * rps, rps)],
                        acc_sh.at[pl.ds(s * rps, rps)])
        plsc.subcore_barrier()

        def gissue(j, slot):
            pltpu.async_copy(tab_sh.at[src_v.at[j]], rows_v.at[slot],
                             gsem.at[slot])

        def gwait(slot):
            pltpu.make_async_copy(tab_sh.at[src_v.at[0]], rows_v.at[slot],
                                  gsem.at[slot]).wait()

        def sissue(j, slot):
            pltpu.async_copy(rows_v.at[slot], acc_sh.at[dst_v.at[j]],
                             ssem.at[slot], add=True)

        def swait(slot):
            pltpu.make_async_copy(rows_v.at[slot], acc_sh.at[dst_v.at[0]],
                                  ssem.at[slot]).wait()

        for b in range(GRP):
            gissue(b, b)

        def body(g, carry):
            h = g % 2
            base = h * GRP
            ob = (1 - h) * GRP
            for b in range(GRP):
                gwait(base + b)
            for b in range(GRP):
                sissue(g * GRP + b, base + b)

            @pl.when(g + 1 < ngrp)
            def _():
                for b in range(GRP):
                    @pl.when(g >= 1)
                    def _():
                        swait(ob + b)
                    gissue((g + 1) * GRP + b, ob + b)
            return carry

        lax.fori_loop(0, ngrp, body, 0)
        for b in range(slots):
            swait(b)
        plsc.subcore_barrier()
        pltpu.sync_copy(acc_sh.at[pl.ds(s * rps, rps)],
                        out_hbm.at[c, pl.ds(s * rps, rps)])

    return k(src_r, dst_r, table)


def _tc_b_call(x_pad, w1, b1, degs, n_pad, hid):
    def body(x_ref, w_ref, b_ref, deg_ref, t1_ref, norm_ref):
        deg = deg_ref[0] + deg_ref[1]
        norm = lax.rsqrt(deg[:, 0:1] + 1.0)
        h1 = jnp.dot(x_ref[...], w_ref[...],
                     preferred_element_type=f32) + b_ref[...]
        norm16 = jnp.broadcast_to(norm, (n_pad, 16))
        t1_ref[...] = jnp.concatenate([h1 * norm, norm16], axis=1)
        norm_ref[...] = norm16

    return pl.pallas_call(
        body,
        out_shape=(
            jax.ShapeDtypeStruct((n_pad, hid + 16), f32),
            jax.ShapeDtypeStruct((n_pad, 16), f32),
        ),
    )(x_pad, w1, b1, degs)


def _tc_c_call(acc1, t1aug, norm16, n_pad, hid):
    def body(acc_ref, t1_ref, norm_ref, t2_ref, s_ref):
        norm = norm_ref[:, 0:1]
        accsum = acc_ref[0] + acc_ref[1]
        agg1 = norm * (accsum[:, :hid] - t1_ref[:, :hid])
        g = jnp.where(agg1 > 0.0,
                      agg1, jnp.exp(jnp.minimum(agg1, 0.0)) - 1.0)
        t2_ref[...] = g * norm
        z = accsum[:, hid:hid + 1] - 2.0 * norm
        s_ref[...] = jnp.broadcast_to(norm * z + norm * norm, norm_ref.shape)

    return pl.pallas_call(
        body,
        out_shape=(
            jax.ShapeDtypeStruct((n_pad, hid), f32),
            jax.ShapeDtypeStruct((n_pad, 16), f32),
        ),
    )(acc1, t1aug, norm16)


def _tc_d_call(acc2, t2, norm16, s16, w2, b2, n, n_pad, f_out):
    def body(acc_ref, t2_ref, norm_ref, s_ref, w_ref, b_ref, out_ref):
        norm = norm_ref[:, 0:1]
        ag = norm * (acc_ref[0] + acc_ref[1] - t2_ref[...])
        h2 = (jnp.dot(ag, w_ref[...], preferred_element_type=f32)
              + s_ref[:, 0:1] * b_ref[...])
        a = h2[:n]
        m = jnp.max(a, axis=1, keepdims=True)
        lse = jnp.log(jnp.sum(jnp.exp(a - m), axis=1, keepdims=True))
        out_ref[...] = a - m - lse

    return pl.pallas_call(
        body,
        out_shape=jax.ShapeDtypeStruct((n, f_out), f32),
    )(acc2, t2, norm16, s16, w2, b2)


def kernel(x, edge_index, W1, b1, W2, b2):
    n, f_in = x.shape
    hid = W1.shape[1]
    f_out = W2.shape[1]
    e = edge_index.shape[1]

    align = NS * 8
    n_pad = ((n + 1 + align - 1) // align) * align
    ch = -(-e // (NW * B))
    ch = max(2 * GRP, ((ch + GRP - 1) // GRP) * GRP)
    e_pad = NW * ch * B

    pad = jnp.full((e_pad - e,), n, dtype=jnp.int32)
    src_r = jnp.concatenate([edge_index[0], pad]).reshape(NW, ch, B)
    dst_r = jnp.concatenate([edge_index[1], pad]).reshape(NW, ch, B)

    x_pad = jnp.zeros((n_pad, f_in), f32).at[:n].set(x)
    zeros = jnp.zeros((n_pad, 16), f32)
    ones = jnp.ones((B, 16), f32)

    degs = _deg_call(dst_r, zeros, ones, n_pad, ch)
    t1aug, norm16 = _tc_b_call(x_pad, W1, b1.reshape(1, hid), degs, n_pad, hid)
    acc1 = _agg_call(src_r, dst_r, t1aug, n_pad, ch, hid + 16)
    t2, s16 = _tc_c_call(acc1, t1aug, norm16, n_pad, hid)
    acc2 = _agg_call(src_r, dst_r, t2, n_pad, ch, hid)
    return _tc_d_call(acc2, t2, norm16, s16, W2, b2.reshape(1, f_out),
                      n, n_pad, f_out)


# speedup vs baseline: 1.9451x; 1.0051x over previous
import functools

import jax
import jax.numpy as jnp
from jax import lax
from jax.experimental import pallas as pl
from jax.experimental.pallas import tpu as pltpu
from jax.experimental.pallas import tpu_sc as plsc

NC = 2
NS = 16
NW = NC * NS
B = 128
GRP = 4

f32 = jnp.float32


def _mesh():
    return plsc.VectorSubcoreMesh(
        core_axis_name="c", subcore_axis_name="s", num_cores=NC, num_subcores=NS
    )


def _deg_call(dst_r, zeros, ones, n_pad, ch):
    rps = n_pad // NS

    @functools.partial(
        pl.kernel,
        out_type=jax.ShapeDtypeStruct((NC, n_pad, 16), f32),
        mesh=_mesh(),
        scratch_types=[
            pltpu.VMEM((ch, B), jnp.int32),
            pltpu.VMEM((B, 16), f32),
            pltpu.VMEM_SHARED((n_pad, 16), f32),
            pltpu.SemaphoreType.DMA,
        ],
        compiler_params=pltpu.CompilerParams(use_tc_tiling_on_sc=False),
    )
    def k(dst_hbm, zeros_hbm, ones_hbm, out_hbm, dst_v, ones_v, acc_sh, dsem):
        c = lax.axis_index("c")
        s = lax.axis_index("s")
        w = c * NS + s
        pltpu.sync_copy(zeros_hbm.at[pl.ds(s * rps, rps)],
                        acc_sh.at[pl.ds(s * rps, rps)])
        pltpu.sync_copy(ones_hbm, ones_v)
        pltpu.sync_copy(dst_hbm.at[w], dst_v)
        plsc.subcore_barrier()

        def body(j, carry):
            pltpu.async_copy(ones_v, acc_sh.at[dst_v.at[j]], dsem, add=True)
            return carry

        lax.fori_loop(0, ch, body, 0)

        def drain(j, carry):
            pltpu.make_async_copy(ones_v, acc_sh.at[dst_v.at[0]], dsem).wait()
            return carry

        lax.fori_loop(0, ch, drain, 0)
        plsc.subcore_barrier()
        pltpu.sync_copy(acc_sh.at[pl.ds(s * rps, rps)],
                        out_hbm.at[c, pl.ds(s * rps, rps)])

    return k(dst_r, zeros, ones)


def _agg_call(src_r, dst_r, table, n_pad, ch, w_feat):
    rps = n_pad // NS
    slots = 2 * GRP
    ngrp = ch // GRP

    @functools.partial(
        pl.kernel,
        out_type=jax.ShapeDtypeStruct((NC, n_pad, w_feat), f32),
        mesh=_mesh(),
        scratch_types=[
            pltpu.VMEM((ch, B), jnp.int32),
            pltpu.VMEM((ch, B), jnp.int32),
            pltpu.VMEM((slots, B, w_feat), f32),
            pltpu.VMEM_SHARED((n_pad, w_feat), f32),
            pltpu.VMEM_SHARED((n_pad, w_feat), f32),
            pltpu.SemaphoreType.DMA((slots,)),
            pltpu.SemaphoreType.DMA((slots,)),
        ],
        compiler_params=pltpu.CompilerParams(use_tc_tiling_on_sc=False),
    )
    def k(src_hbm, dst_hbm, table_hbm, out_hbm, src_v, dst_v, rows_v, acc_sh,
          tab_sh, gsem, ssem):
        c = lax.axis_index("c")
        s = lax.axis_index("s")
        w = c * NS + s
        pltpu.sync_copy(src_hbm.at[w], src_v)
        pltpu.sync_copy(dst_hbm.at[w], dst_v)
        pltpu.sync_copy(table_hbm.at[pl.ds(s * rps, rps)],
                        tab_sh.at[pl.ds(s * rps, rps)])
        pltpu.sync_copy(table_hbm.at[pl.ds(s * rps, rps)],
                        acc_sh.at[pl.ds(s * rps, rps)])
        plsc.subcore_barrier()

        def gissue(j, slot):
            pltpu.async_copy(tab_sh.at[src_v.at[j]], rows_v.at[slot],
                             gsem.at[slot])

        def gwait(slot):
            pltpu.make_async_copy(tab_sh.at[src_v.at[0]], rows_v.at[slot],
                                  gsem.at[slot]).wait()

        def sissue(j, slot):
            pltpu.async_copy(rows_v.at[slot], acc_sh.at[dst_v.at[j]],
                             ssem.at[slot], add=True)

        def swait(slot):
            pltpu.make_async_copy(rows_v.at[slot], acc_sh.at[dst_v.at[0]],
                                  ssem.at[slot]).wait()

        for b in range(GRP):
            gissue(b, b)

        def body(g, carry):
            h = g % 2
            base = h * GRP
            ob = (1 - h) * GRP
            for b in range(GRP):
                gwait(base + b)
            for b in range(GRP):
                sissue(g * GRP + b, base + b)

            @pl.when(g + 1 < ngrp)
            def _():
                for b in range(GRP):
                    @pl.when(g >= 1)
                    def _():
                        swait(ob + b)
                    gissue((g + 1) * GRP + b, ob + b)
            return carry

        lax.fori_loop(0, ngrp, body, 0)
        for b in range(slots):
            swait(b)
        plsc.subcore_barrier()
        pltpu.sync_copy(acc_sh.at[pl.ds(s * rps, rps)],
                        out_hbm.at[c, pl.ds(s * rps, rps)])

    return k(src_r, dst_r, table)


def _tc_b1_call(x, w1, b1, n, hid):
    def body(x_ref, w_ref, b_ref, h1_ref):
        h1_ref[...] = jnp.dot(x_ref[...], w_ref[...],
                              preferred_element_type=f32) + b_ref[...]

    return pl.pallas_call(
        body,
        out_shape=jax.ShapeDtypeStruct((n, hid), f32),
    )(x, w1, b1)


def _tc_b2_call(h1, degs, n, n_pad, hid):
    def body(h1_ref, deg_ref, t1_ref, norm_ref):
        deg = deg_ref[0] + deg_ref[1]
        norm = lax.rsqrt(deg[:, 0:1] + 1.0)
        norm16 = jnp.broadcast_to(norm, (n_pad, 16))
        t1_ref[:n] = jnp.concatenate(
            [h1_ref[...] * norm[:n], norm16[:n]], axis=1)
        t1_ref[n:] = jnp.zeros((n_pad - n, hid + 16), f32)
        norm_ref[...] = norm16

    return pl.pallas_call(
        body,
        out_shape=(
            jax.ShapeDtypeStruct((n_pad, hid + 16), f32),
            jax.ShapeDtypeStruct((n_pad, 16), f32),
        ),
    )(h1, degs)


def _tc_c_call(acc1, t1aug, norm16, n_pad, hid):
    def body(acc_ref, t1_ref, norm_ref, t2_ref, s_ref):
        norm = norm_ref[:, 0:1]
        accsum = acc_ref[0] + acc_ref[1]
        agg1 = norm * (accsum[:, :hid] - t1_ref[:, :hid])
        g = jnp.where(agg1 > 0.0,
                      agg1, jnp.exp(jnp.minimum(agg1, 0.0)) - 1.0)
        t2_ref[...] = g * norm
        z = accsum[:, hid:hid + 1] - 2.0 * norm
        s_ref[...] = jnp.broadcast_to(norm * z + norm * norm, norm_ref.shape)

    return pl.pallas_call(
        body,
        out_shape=(
            jax.ShapeDtypeStruct((n_pad, hid), f32),
            jax.ShapeDtypeStruct((n_pad, 16), f32),
        ),
    )(acc1, t1aug, norm16)


def _tc_d_call(acc2, t2, norm16, s16, w2, b2, n, n_pad, f_out):
    def body(acc_ref, t2_ref, norm_ref, s_ref, w_ref, b_ref, out_ref):
        norm = norm_ref[:, 0:1]
        ag = norm * (acc_ref[0] + acc_ref[1] - t2_ref[...])
        h2 = (jnp.dot(ag, w_ref[...], preferred_element_type=f32)
              + s_ref[:, 0:1] * b_ref[...])
        a = h2[:n]
        m = jnp.max(a, axis=1, keepdims=True)
        lse = jnp.log(jnp.sum(jnp.exp(a - m), axis=1, keepdims=True))
        out_ref[...] = a - m - lse

    return pl.pallas_call(
        body,
        out_shape=jax.ShapeDtypeStruct((n, f_out), f32),
    )(acc2, t2, norm16, s16, w2, b2)


def kernel(x, edge_index, W1, b1, W2, b2):
    n, f_in = x.shape
    hid = W1.shape[1]
    f_out = W2.shape[1]
    e = edge_index.shape[1]

    align = NS * 8
    n_pad = ((n + 1 + align - 1) // align) * align
    ch = -(-e // (NW * B))
    ch = max(2 * GRP, ((ch + GRP - 1) // GRP) * GRP)
    e_pad = NW * ch * B

    pad = jnp.full((e_pad - e,), n, dtype=jnp.int32)
    src_r = jnp.concatenate([edge_index[0], pad]).reshape(NW, ch, B)
    dst_r = jnp.concatenate([edge_index[1], pad]).reshape(NW, ch, B)

    zeros = jnp.zeros((n_pad, 16), f32)
    ones = jnp.ones((B, 16), f32)

    h1 = _tc_b1_call(x, W1, b1.reshape(1, hid), n, hid)
    degs = _deg_call(dst_r, zeros, ones, n_pad, ch)
    t1aug, norm16 = _tc_b2_call(h1, degs, n, n_pad, hid)
    acc1 = _agg_call(src_r, dst_r, t1aug, n_pad, ch, hid + 16)
    t2, s16 = _tc_c_call(acc1, t1aug, norm16, n_pad, hid)
    acc2 = _agg_call(src_r, dst_r, t2, n_pad, ch, hid)
    return _tc_d_call(acc2, t2, norm16, s16, W2, b2.reshape(1, f_out),
                      n, n_pad, f_out)


# speedup vs baseline: 2.0605x; 1.0593x over previous
import functools

import jax
import jax.numpy as jnp
from jax import lax
from jax.experimental import pallas as pl
from jax.experimental.pallas import tpu as pltpu
from jax.experimental.pallas import tpu_sc as plsc

NC = 2
NS = 16
NW = NC * NS
B = 128
GRP = 4

f32 = jnp.float32


def _mesh():
    return plsc.VectorSubcoreMesh(
        core_axis_name="c", subcore_axis_name="s", num_cores=NC, num_subcores=NS
    )


def _fetch_idx(ei_hbm, row, w, epw, buf, n):
    pltpu.sync_copy(ei_hbm.at[row, pl.ds(w * epw, epw)], buf.at[pl.ds(0, epw)])
    total = buf.shape[0]
    lanes = lax.iota(jnp.int32, 16)
    for i in range((total - epw) // 16):
        buf[pl.ds(epw + i * 16, 16)] = n + lanes + 16 * (i % 7)


def _deg_call(ei, zeros, ones, n, epw, n_pad, ch):
    rps = n_pad // NS

    @functools.partial(
        pl.kernel,
        out_type=jax.ShapeDtypeStruct((NC, n_pad, 16), f32),
        mesh=_mesh(),
        scratch_types=[
            pltpu.VMEM((ch * B,), jnp.int32),
            pltpu.VMEM((B, 16), f32),
            pltpu.VMEM_SHARED((n_pad, 16), f32),
            pltpu.SemaphoreType.DMA,
        ],
        compiler_params=pltpu.CompilerParams(use_tc_tiling_on_sc=False),
    )
    def k(ei_hbm, zeros_hbm, ones_hbm, out_hbm, dst_v, ones_v, acc_sh, dsem):
        c = lax.axis_index("c")
        s = lax.axis_index("s")
        w = c * NS + s
        pltpu.sync_copy(zeros_hbm.at[pl.ds(s * rps, rps)],
                        acc_sh.at[pl.ds(s * rps, rps)])
        pltpu.sync_copy(ones_hbm, ones_v)
        _fetch_idx(ei_hbm, 1, w, epw, dst_v, n)
        plsc.subcore_barrier()

        def body(j, carry):
            pltpu.async_copy(ones_v, acc_sh.at[dst_v.at[pl.ds(j * B, B)]],
                             dsem, add=True)
            return carry

        lax.fori_loop(0, ch, body, 0)

        def drain(j, carry):
            pltpu.make_async_copy(ones_v, acc_sh.at[dst_v.at[pl.ds(0, B)]],
                                  dsem).wait()
            return carry

        lax.fori_loop(0, ch, drain, 0)
        plsc.subcore_barrier()
        pltpu.sync_copy(acc_sh.at[pl.ds(s * rps, rps)],
                        out_hbm.at[c, pl.ds(s * rps, rps)])

    return k(ei, zeros, ones)


def _agg_call(ei, table, n, epw, n_pad, ch, w_feat):
    rps = n_pad // NS
    slots = 2 * GRP
    ngrp = ch // GRP

    @functools.partial(
        pl.kernel,
        out_type=jax.ShapeDtypeStruct((NC, n_pad, w_feat), f32),
        mesh=_mesh(),
        scratch_types=[
            pltpu.VMEM((ch * B,), jnp.int32),
            pltpu.VMEM((ch * B,), jnp.int32),
            pltpu.VMEM((slots, B, w_feat), f32),
            pltpu.VMEM_SHARED((n_pad, w_feat), f32),
            pltpu.VMEM_SHARED((n_pad, w_feat), f32),
            pltpu.SemaphoreType.DMA((slots,)),
            pltpu.SemaphoreType.DMA((slots,)),
        ],
        compiler_params=pltpu.CompilerParams(use_tc_tiling_on_sc=False),
    )
    def k(ei_hbm, table_hbm, out_hbm, src_v, dst_v, rows_v, acc_sh, tab_sh,
          gsem, ssem):
        c = lax.axis_index("c")
        s = lax.axis_index("s")
        w = c * NS + s
        _fetch_idx(ei_hbm, 0, w, epw, src_v, n)
        _fetch_idx(ei_hbm, 1, w, epw, dst_v, n)
        pltpu.sync_copy(table_hbm.at[pl.ds(s * rps, rps)],
                        tab_sh.at[pl.ds(s * rps, rps)])
        pltpu.sync_copy(table_hbm.at[pl.ds(s * rps, rps)],
                        acc_sh.at[pl.ds(s * rps, rps)])
        plsc.subcore_barrier()

        def gissue(j, slot):
            pltpu.async_copy(tab_sh.at[src_v.at[pl.ds(j * B, B)]],
                             rows_v.at[slot], gsem.at[slot])

        def gwait(slot):
            pltpu.make_async_copy(tab_sh.at[src_v.at[pl.ds(0, B)]],
                                  rows_v.at[slot], gsem.at[slot]).wait()

        def sissue(j, slot):
            pltpu.async_copy(rows_v.at[slot],
                             acc_sh.at[dst_v.at[pl.ds(j * B, B)]],
                             ssem.at[slot], add=True)

        def swait(slot):
            pltpu.make_async_copy(rows_v.at[slot],
                                  acc_sh.at[dst_v.at[pl.ds(0, B)]],
                                  ssem.at[slot]).wait()

        for b in range(GRP):
            gissue(b, b)

        def body(g, carry):
            h = g % 2
            base = h * GRP
            ob = (1 - h) * GRP
            for b in range(GRP):
                gwait(base + b)
            for b in range(GRP):
                sissue(g * GRP + b, base + b)

            @pl.when(g + 1 < ngrp)
            def _():
                for b in range(GRP):
                    @pl.when(g >= 1)
                    def _():
                        swait(ob + b)
                    gissue((g + 1) * GRP + b, ob + b)
            return carry

        lax.fori_loop(0, ngrp, body, 0)
        for b in range(slots):
            swait(b)
        plsc.subcore_barrier()
        pltpu.sync_copy(acc_sh.at[pl.ds(s * rps, rps)],
                        out_hbm.at[c, pl.ds(s * rps, rps)])

    return k(ei, table)


def _tc_b1_call(x, w1, b1, n, n_pad, hid):
    def body(x_ref, w_ref, b_ref, h1_ref):
        h1_ref[:n] = jnp.dot(x_ref[...], w_ref[...],
                             preferred_element_type=f32) + b_ref[...]
        h1_ref[n:] = jnp.zeros((n_pad - n, hid), f32)

    return pl.pallas_call(
        body,
        out_shape=jax.ShapeDtypeStruct((n_pad, hid), f32),
    )(x, w1, b1)


def _tc_b2_call(h1, degs, n_pad, hid, blk):
    grid = n_pad // blk

    def body(h1_ref, deg_ref, t1_ref, norm_ref):
        deg = deg_ref[0] + deg_ref[1]
        norm = lax.rsqrt(deg[:, 0:1] + 1.0)
        norm16 = jnp.broadcast_to(norm, (blk, 16))
        t1_ref[...] = jnp.concatenate([h1_ref[...] * norm, norm16], axis=1)
        norm_ref[...] = norm16

    return pl.pallas_call(
        body,
        grid=(grid,),
        in_specs=[
            pl.BlockSpec((blk, hid), lambda i: (i, 0)),
            pl.BlockSpec((2, blk, 16), lambda i: (0, i, 0)),
        ],
        out_specs=(
            pl.BlockSpec((blk, hid + 16), lambda i: (i, 0)),
            pl.BlockSpec((blk, 16), lambda i: (i, 0)),
        ),
        out_shape=(
            jax.ShapeDtypeStruct((n_pad, hid + 16), f32),
            jax.ShapeDtypeStruct((n_pad, 16), f32),
        ),
    )(h1, degs)


def _tc_c_call(acc1, t1aug, norm16, n_pad, hid, blk):
    grid = n_pad // blk

    def body(acc_ref, t1_ref, norm_ref, t2_ref, s_ref):
        norm = norm_ref[:, 0:1]
        accsum = acc_ref[0] + acc_ref[1]
        agg1 = norm * (accsum[:, :hid] - t1_ref[:, :hid])
        g = jnp.where(agg1 > 0.0,
                      agg1, jnp.exp(jnp.minimum(agg1, 0.0)) - 1.0)
        t2_ref[...] = g * norm
        z = accsum[:, hid:hid + 1] - 2.0 * norm
        s_ref[...] = jnp.broadcast_to(norm * z + norm * norm, (blk, 16))

    return pl.pallas_call(
        body,
        grid=(grid,),
        in_specs=[
            pl.BlockSpec((2, blk, hid + 16), lambda i: (0, i, 0)),
            pl.BlockSpec((blk, hid + 16), lambda i: (i, 0)),
            pl.BlockSpec((blk, 16), lambda i: (i, 0)),
        ],
        out_specs=(
            pl.BlockSpec((blk, hid), lambda i: (i, 0)),
            pl.BlockSpec((blk, 16), lambda i: (i, 0)),
        ),
        out_shape=(
            jax.ShapeDtypeStruct((n_pad, hid), f32),
            jax.ShapeDtypeStruct((n_pad, 16), f32),
        ),
    )(acc1, t1aug, norm16)


def _tc_d_call(acc2, t2, norm16, s16, w2, b2, n, f_out, hid, blk):
    grid = n // blk

    def body(acc_ref, t2_ref, norm_ref, s_ref, w_ref, b_ref, out_ref):
        norm = norm_ref[:, 0:1]
        ag = norm * (acc_ref[0] + acc_ref[1] - t2_ref[...])
        a = (jnp.dot(ag, w_ref[...], preferred_element_type=f32)
             + s_ref[:, 0:1] * b_ref[...])
        m = jnp.max(a, axis=1, keepdims=True)
        lse = jnp.log(jnp.sum(jnp.exp(a - m), axis=1, keepdims=True))
        out_ref[...] = a - m - lse

    return pl.pallas_call(
        body,
        grid=(grid,),
        in_specs=[
            pl.BlockSpec((2, blk, hid), lambda i: (0, i, 0)),
            pl.BlockSpec((blk, hid), lambda i: (i, 0)),
            pl.BlockSpec((blk, 16), lambda i: (i, 0)),
            pl.BlockSpec((blk, 16), lambda i: (i, 0)),
            pl.BlockSpec((hid, f_out), lambda i: (0, 0)),
            pl.BlockSpec((1, f_out), lambda i: (0, 0)),
        ],
        out_specs=pl.BlockSpec((blk, f_out), lambda i: (i, 0)),
        out_shape=jax.ShapeDtypeStruct((n, f_out), f32),
    )(acc2, t2, norm16, s16, w2, b2)


def kernel(x, edge_index, W1, b1, W2, b2):
    n, f_in = x.shape
    hid = W1.shape[1]
    f_out = W2.shape[1]
    e = edge_index.shape[1]

    align = NS * 8
    n_pad = ((n + 1 + align - 1) // align) * align
    epw = e // NW
    ch = -(-epw // B)
    ch = max(2 * GRP, ((ch + GRP - 1) // GRP) * GRP)

    zeros = jnp.zeros((n_pad, 16), f32)
    ones = jnp.ones((B, 16), f32)

    h1 = _tc_b1_call(x, W1, b1.reshape(1, hid), n, n_pad, hid)
    degs = _deg_call(edge_index, zeros, ones, n, epw, n_pad, ch)
    t1aug, norm16 = _tc_b2_call(h1, degs, n_pad, hid, n_pad // 16)
    acc1 = _agg_call(edge_index, t1aug, n, epw, n_pad, ch, hid + 16)
    t2, s16 = _tc_c_call(acc1, t1aug, norm16, n_pad, hid, n_pad // 16)
    acc2 = _agg_call(edge_index, t2, n, epw, n_pad, ch, hid)
    return _tc_d_call(acc2, t2, norm16, s16, W2, b2.reshape(1, f_out),
                      n, f_out, hid, n // 10)


# speedup vs baseline: 2.1376x; 1.0374x over previous
import functools

import jax
import jax.numpy as jnp
from jax import lax
from jax.experimental import pallas as pl
from jax.experimental.pallas import tpu as pltpu
from jax.experimental.pallas import tpu_sc as plsc

NC = 2
NS = 16
NW = NC * NS
B = 128
GRP = 4

f32 = jnp.float32


def _mesh():
    return plsc.VectorSubcoreMesh(
        core_axis_name="c", subcore_axis_name="s", num_cores=NC, num_subcores=NS
    )


def _fetch_idx(ei_hbm, row, w, epw, buf, n):
    pltpu.sync_copy(ei_hbm.at[row, pl.ds(w * epw, epw)], buf.at[pl.ds(0, epw)])
    total = buf.shape[0]
    lanes = lax.iota(jnp.int32, 16)
    for i in range((total - epw) // 16):
        buf[pl.ds(epw + i * 16, 16)] = n + lanes + 16 * (i % 7)


def _deg_call(ei, zeros, ones, n, epw, n_pad, ch):
    rps = n_pad // NS

    @functools.partial(
        pl.kernel,
        out_type=jax.ShapeDtypeStruct((NC, n_pad, 16), f32),
        mesh=_mesh(),
        scratch_types=[
            pltpu.VMEM((ch * B,), jnp.int32),
            pltpu.VMEM((B, 16), f32),
            pltpu.VMEM_SHARED((n_pad, 16), f32),
            pltpu.SemaphoreType.DMA,
        ],
        compiler_params=pltpu.CompilerParams(use_tc_tiling_on_sc=False),
    )
    def k(ei_hbm, zeros_hbm, ones_hbm, out_hbm, dst_v, ones_v, acc_sh, dsem):
        c = lax.axis_index("c")
        s = lax.axis_index("s")
        w = c * NS + s
        pltpu.sync_copy(zeros_hbm.at[pl.ds(s * rps, rps)],
                        acc_sh.at[pl.ds(s * rps, rps)])
        pltpu.sync_copy(ones_hbm, ones_v)
        _fetch_idx(ei_hbm, 1, w, epw, dst_v, n)
        plsc.subcore_barrier()

        def body(j, carry):
            pltpu.async_copy(ones_v, acc_sh.at[dst_v.at[pl.ds(j * B, B)]],
                             dsem, add=True)
            return carry

        lax.fori_loop(0, ch, body, 0)

        def drain(j, carry):
            pltpu.make_async_copy(ones_v, acc_sh.at[dst_v.at[pl.ds(0, B)]],
                                  dsem).wait()
            return carry

        lax.fori_loop(0, ch, drain, 0)
        plsc.subcore_barrier()
        pltpu.sync_copy(acc_sh.at[pl.ds(s * rps, rps)],
                        out_hbm.at[c, pl.ds(s * rps, rps)])

    return k(ei, zeros, ones)


def _agg_call(ei, table, n, epw, n_pad, ch, w_feat):
    rps = n_pad // NS
    slots = 2 * GRP
    ngrp = ch // GRP

    @functools.partial(
        pl.kernel,
        out_type=jax.ShapeDtypeStruct((NC, n_pad, w_feat), f32),
        mesh=_mesh(),
        scratch_types=[
            pltpu.VMEM((ch * B,), jnp.int32),
            pltpu.VMEM((ch * B,), jnp.int32),
            pltpu.VMEM((slots, B, w_feat), f32),
            pltpu.VMEM_SHARED((n_pad, w_feat), f32),
            pltpu.VMEM_SHARED((n_pad, w_feat), f32),
            pltpu.SemaphoreType.DMA((slots,)),
            pltpu.SemaphoreType.DMA((slots,)),
        ],
        compiler_params=pltpu.CompilerParams(use_tc_tiling_on_sc=False),
    )
    def k(ei_hbm, table_hbm, out_hbm, src_v, dst_v, rows_v, acc_sh, tab_sh,
          gsem, ssem):
        c = lax.axis_index("c")
        s = lax.axis_index("s")
        w = c * NS + s
        _fetch_idx(ei_hbm, 0, w, epw, src_v, n)
        _fetch_idx(ei_hbm, 1, w, epw, dst_v, n)
        pltpu.sync_copy(table_hbm.at[pl.ds(s * rps, rps)],
                        tab_sh.at[pl.ds(s * rps, rps)])
        pltpu.sync_copy(table_hbm.at[pl.ds(s * rps, rps)],
                        acc_sh.at[pl.ds(s * rps, rps)])
        plsc.subcore_barrier()

        def gissue(j, slot):
            pltpu.async_copy(tab_sh.at[src_v.at[pl.ds(j * B, B)]],
                             rows_v.at[slot], gsem.at[slot])

        def gwait(slot):
            pltpu.make_async_copy(tab_sh.at[src_v.at[pl.ds(0, B)]],
                                  rows_v.at[slot], gsem.at[slot]).wait()

        def sissue(j, slot):
            pltpu.async_copy(rows_v.at[slot],
                             acc_sh.at[dst_v.at[pl.ds(j * B, B)]],
                             ssem.at[slot], add=True)

        def swait(slot):
            pltpu.make_async_copy(rows_v.at[slot],
                                  acc_sh.at[dst_v.at[pl.ds(0, B)]],
                                  ssem.at[slot]).wait()

        for b in range(GRP):
            gissue(b, b)

        def body(g, carry):
            h = g % 2
            base = h * GRP
            ob = (1 - h) * GRP
            for b in range(GRP):
                gwait(base + b)
            for b in range(GRP):
                sissue(g * GRP + b, base + b)

            @pl.when(g + 1 < ngrp)
            def _():
                for b in range(GRP):
                    @pl.when(g >= 1)
                    def _():
                        swait(ob + b)
                    gissue((g + 1) * GRP + b, ob + b)
            return carry

        lax.fori_loop(0, ngrp, body, 0)
        for b in range(slots):
            swait(b)
        plsc.subcore_barrier()
        pltpu.sync_copy(acc_sh.at[pl.ds(s * rps, rps)],
                        out_hbm.at[c, pl.ds(s * rps, rps)])

    return k(ei, table)


def _tc_b1_call(x, w1, b1, n, n_pad, hid):
    def body(x_ref, w_ref, b_ref, h1_ref):
        h1_ref[:n] = jnp.dot(x_ref[...], w_ref[...],
                             preferred_element_type=f32) + b_ref[...]
        h1_ref[n:] = jnp.zeros((n_pad - n, hid), f32)

    return pl.pallas_call(
        body,
        out_shape=jax.ShapeDtypeStruct((n_pad, hid), f32),
    )(x, w1, b1)


def _tc_b2_call(h1, degs, n_pad, hid, blk):
    blk = n_pad

    def body(h1_ref, deg_ref, t1_ref, norm_ref):
        deg = deg_ref[0] + deg_ref[1]
        norm = lax.rsqrt(deg[:, 0:1] + 1.0)
        norm16 = jnp.broadcast_to(norm, (blk, 16))
        t1_ref[...] = jnp.concatenate([h1_ref[...] * norm, norm16], axis=1)
        norm_ref[...] = norm16

    return pl.pallas_call(
        body,
        out_shape=(
            jax.ShapeDtypeStruct((n_pad, hid + 16), f32),
            jax.ShapeDtypeStruct((n_pad, 16), f32),
        ),
    )(h1, degs)


def _tc_c_call(acc1, t1aug, norm16, n_pad, hid, blk):
    blk = n_pad

    def body(acc_ref, t1_ref, norm_ref, t2_ref, s_ref):
        norm = norm_ref[:, 0:1]
        accsum = acc_ref[0] + acc_ref[1]
        agg1 = norm * (accsum[:, :hid] - t1_ref[:, :hid])
        g = jnp.where(agg1 > 0.0,
                      agg1, jnp.exp(jnp.minimum(agg1, 0.0)) - 1.0)
        t2_ref[...] = g * norm
        z = accsum[:, hid:hid + 1] - 2.0 * norm
        s_ref[...] = jnp.broadcast_to(norm * z + norm * norm, (blk, 16))

    return pl.pallas_call(
        body,
        out_shape=(
            jax.ShapeDtypeStruct((n_pad, hid), f32),
            jax.ShapeDtypeStruct((n_pad, 16), f32),
        ),
    )(acc1, t1aug, norm16)


def _tc_d_call(acc2, t2, norm16, s16, w2, b2, n, f_out, hid, blk):
    def body(acc_ref, t2_ref, norm_ref, s_ref, w_ref, b_ref, out_ref):
        norm = norm_ref[:, 0:1]
        ag = norm * (acc_ref[0] + acc_ref[1] - t2_ref[...])
        a = (jnp.dot(ag[:n], w_ref[...], preferred_element_type=f32)
             + s_ref[:n, 0:1] * b_ref[...])
        m = jnp.max(a, axis=1, keepdims=True)
        lse = jnp.log(jnp.sum(jnp.exp(a - m), axis=1, keepdims=True))
        out_ref[...] = a - m - lse

    return pl.pallas_call(
        body,
        out_shape=jax.ShapeDtypeStruct((n, f_out), f32),
    )(acc2, t2, norm16, s16, w2, b2)


def kernel(x, edge_index, W1, b1, W2, b2):
    n, f_in = x.shape
    hid = W1.shape[1]
    f_out = W2.shape[1]
    e = edge_index.shape[1]

    align = NS * 8
    n_pad = ((n + 1 + align - 1) // align) * align
    epw = e // NW
    ch = -(-epw // B)
    ch = max(2 * GRP, ((ch + GRP - 1) // GRP) * GRP)

    zeros = jnp.zeros((n_pad, 16), f32)
    ones = jnp.ones((B, 16), f32)

    h1 = _tc_b1_call(x, W1, b1.reshape(1, hid), n, n_pad, hid)
    degs = _deg_call(edge_index, zeros, ones, n, epw, n_pad, ch)
    t1aug, norm16 = _tc_b2_call(h1, degs, n_pad, hid, n_pad // 16)
    acc1 = _agg_call(edge_index, t1aug, n, epw, n_pad, ch, hid + 16)
    t2, s16 = _tc_c_call(acc1, t1aug, norm16, n_pad, hid, n_pad // 16)
    acc2 = _agg_call(edge_index, t2, n, epw, n_pad, ch, hid)
    return _tc_d_call(acc2, t2, norm16, s16, W2, b2.reshape(1, f_out),
                      n, f_out, hid, n // 10)
